# baseline (speedup 1.0000x reference)
import functools

import jax
import jax.numpy as jnp
from jax import lax
from jax.experimental import pallas as pl
from jax.experimental.pallas import tpu as pltpu
from jax.experimental.pallas import tpu_sc as plsc

_NUM_CLASSES = 1000000
_HIDDEN = 64
_BATCH = 16384

_INFO = plsc.get_sparse_core_info()
_NC = _INFO.num_cores
_NS = _INFO.num_subcores
_L = _INFO.num_lanes
_NW = _NC * _NS
_NBLK = (_NUM_CLASSES + 1 + 127) // 128
_NB_W = (_NBLK + _NW - 1) // _NW
_NGRP = _BATCH // _L
_LIST = _BATCH + _L
_DUMP = _BATCH

_mesh = plsc.VectorSubcoreMesh(core_axis_name="c", subcore_axis_name="s")


@functools.partial(
    pl.kernel,
    mesh=_mesh,
    out_type=jax.ShapeDtypeStruct((_BATCH + 8, 2 * _HIDDEN), jnp.float32),
    scratch_types=[
        pltpu.VMEM((_BATCH,), jnp.int32),
        pltpu.VMEM((_BATCH,), jnp.int32),
        pltpu.VMEM((_LIST,), jnp.int32),
        pltpu.VMEM((_LIST,), jnp.int32),
        pltpu.VMEM((_HIDDEN, 128), jnp.float32),
        pltpu.VMEM((_L, 2 * _HIDDEN), jnp.float32),
        pltpu.VMEM((8, _L), jnp.int32),
        pltpu.SemaphoreType.DMA,
    ],
    compiler_params=pltpu.CompilerParams(needs_layout_passes=False),
)
def _embed(labels_hbm, drop_hbm, tab_t_hbm, tail_hbm, out_hbm,
           lbl_v, drop_v, mylbl_v, mypos_v, blk_v, stage_v, posr_v, sem):
    wid = lax.axis_index("s") * _NC + lax.axis_index("c")
    iota = lax.iota(jnp.int32, _L)

    pltpu.sync_copy(labels_hbm, lbl_v)
    pltpu.sync_copy(drop_hbm, drop_v)

    def scan_group(g, cnt):
        sl = pl.ds(g * _L, _L)
        sel = jnp.where(drop_v[sl] != 0, _NUM_CLASSES, lbl_v[sl])
        blk = sel >> 7
        mine = (blk // _NB_W) == wid
        plsc.store_compressed(mylbl_v.at[pl.ds(cnt, _L)], sel, mask=mine)
        plsc.store_compressed(mypos_v.at[pl.ds(cnt, _L)],
                              iota + g * _L, mask=mine)
        return cnt + jnp.sum(mine.astype(jnp.int32), axis=0)

    cnt = lax.fori_loop(0, _NGRP, scan_group, jnp.int32(0))
    ngrp_mine = (cnt + _L - 1) // _L

    def do_block(ci, _):
        c = wid * _NB_W + ci

        @pl.when(c < _NBLK - 1)
        def _fetch_main():
            off = pl.multiple_of(c * 128, 128)
            pltpu.sync_copy(tab_t_hbm.at[:, pl.ds(off, 128)], blk_v)

        @pl.when(c == _NBLK - 1)
        def _fetch_tail():
            pltpu.sync_copy(tail_hbm, blk_v)

        @pl.when(c < _NBLK)
        def _scan_matches():
            def do_group(gi, _):
                sl = pl.ds(gi * _L, _L)
                l16 = mylbl_v[sl]
                p16 = mypos_v[sl]
                valid = ((iota + gi * _L) < cnt) & ((l16 >> 7) == c)
                nhit = jnp.sum(valid.astype(jnp.int32), axis=0)

                @pl.when(nhit > 0)
                def _extract():
                    lane = l16 & 127
                    pos = jnp.where(valid, p16, _DUMP)
                    prow = posr_v.at[0]
                    prow[pl.ds(0, _L)] = pos
                    for h in range(_HIDDEN):
                        hv = jnp.full((_L,), h, jnp.int32)
                        x = plsc.load_gather(blk_v, [hv, lane])
                        plsc.store_scatter(stage_v, [iota, hv], x)
                    pltpu.async_copy(
                        stage_v, out_hbm.at[posr_v.at[0]], sem
                    ).wait()

                return 0

            lax.fori_loop(0, ngrp_mine, do_group, 0)

        return 0

    lax.fori_loop(0, _NB_W, do_block, 0)


def kernel(labels, force_drop_ids, embedding_table):
    lbl = labels.astype(jnp.int32)
    drop = force_drop_ids.astype(jnp.int32)
    tab_t = embedding_table.T
    tail_start = (_NBLK - 1) * 128
    tail = jnp.pad(tab_t[:, tail_start:],
                   ((0, 0), (0, _NBLK * 128 - (_NUM_CLASSES + 1))))
    out128 = _embed(lbl, drop, tab_t, tail)
    return out128[:_BATCH, :_HIDDEN]


# speedup vs baseline: 28.1991x; 28.1991x over previous
import functools

import jax
import jax.numpy as jnp
from jax import lax
from jax.experimental import pallas as pl
from jax.experimental.pallas import tpu as pltpu
from jax.experimental.pallas import tpu_sc as plsc

_NUM_CLASSES = 1000000
_HIDDEN = 64
_BATCH = 16384

_INFO = plsc.get_sparse_core_info()
_NC = _INFO.num_cores
_NS = _INFO.num_subcores
_L = _INFO.num_lanes
_NW = _NC * _NS
_B_PER_W = _BATCH // _NW
_NBLK = (_NUM_CLASSES + 1 + 127) // 128

_mesh = plsc.VectorSubcoreMesh(core_axis_name="c", subcore_axis_name="s")


@functools.partial(
    pl.kernel,
    mesh=_mesh,
    out_type=jax.ShapeDtypeStruct((_BATCH, _HIDDEN), jnp.float32),
    scratch_types=[
        pltpu.SMEM((_B_PER_W,), jnp.int32),
        pltpu.SMEM((_B_PER_W,), jnp.int32),
        pltpu.VMEM_SHARED((_NW, _B_PER_W), jnp.int32),
        pltpu.VMEM_SHARED((_NW, _B_PER_W), jnp.int32),
        pltpu.VMEM((2, _HIDDEN, 128), jnp.float32),
        pltpu.VMEM((_B_PER_W, _HIDDEN), jnp.float32),
        pltpu.SemaphoreType.DMA,
        pltpu.SemaphoreType.DMA,
    ],
    compiler_params=pltpu.CompilerParams(needs_layout_passes=False),
)
def _embed(labels_hbm, drop_hbm, tab_t_hbm, tail_hbm, out_hbm,
           lbl_s, drop_s, lbl_sp, drop_sp, ring_v, out_v, sem0, sem1):
    wid = lax.axis_index("s") * _NC + lax.axis_index("c")
    base = wid * _B_PER_W
    iota = lax.iota(jnp.int32, _L)
    sems = (sem0, sem1)

    pltpu.sync_copy(labels_hbm.at[pl.ds(base, _B_PER_W)], lbl_sp.at[wid])
    pltpu.sync_copy(drop_hbm.at[pl.ds(base, _B_PER_W)], drop_sp.at[wid])
    pltpu.sync_copy(lbl_sp.at[wid], lbl_s)
    pltpu.sync_copy(drop_sp.at[wid], drop_s)

    def selected(i):
        return lax.select(drop_s[i] != 0, _NUM_CLASSES, lbl_s[i])

    def issue_fetch(i, par):
        r = selected(i)
        c = r >> 7
        slot = ring_v.at[par]

        @pl.when(c < _NBLK - 1)
        def _main():
            off = pl.multiple_of(c * 128, 128)
            pltpu.async_copy(tab_t_hbm.at[:, pl.ds(off, 128)], slot,
                             sems[par])

        @pl.when(c == _NBLK - 1)
        def _tail():
            pltpu.async_copy(tail_hbm, slot, sems[par])

    def wait_fetch(par):
        pltpu.make_async_copy(tail_hbm, ring_v.at[par], sems[par]).wait()

    def extract(i, par):
        r = selected(i)
        lane = jnp.full((_L,), r & 127, jnp.int32)
        slot = ring_v.at[par]
        orow = out_v.at[i]
        for q in range(_HIDDEN // _L):
            hv = iota + q * _L
            x = plsc.load_gather(slot, [hv, lane])
            orow[pl.ds(q * _L, _L)] = x

    issue_fetch(0, 0)
    issue_fetch(1, 1)

    def body(g, _):
        i = g * 2
        for par in range(2):
            wait_fetch(par)
            extract(i + par, par)

            @pl.when(i + par + 2 < _B_PER_W)
            def _next():
                issue_fetch(i + par + 2, par)

        return 0

    lax.fori_loop(0, _B_PER_W // 2, body, 0)
    pltpu.sync_copy(out_v, out_hbm.at[pl.ds(base, _B_PER_W)])


def kernel(labels, force_drop_ids, embedding_table):
    lbl = labels.astype(jnp.int32)
    drop = force_drop_ids.astype(jnp.int32)
    tab_t = embedding_table.T
    tail_start = (_NBLK - 1) * 128
    tail = jnp.pad(tab_t[:, tail_start:],
                   ((0, 0), (0, _NBLK * 128 - (_NUM_CLASSES + 1))))
    return _embed(lbl, drop, tab_t, tail)


# speedup vs baseline: 38.1496x; 1.3529x over previous
import functools

import jax
import jax.numpy as jnp
from jax import lax
from jax.experimental import pallas as pl
from jax.experimental.pallas import tpu as pltpu
from jax.experimental.pallas import tpu_sc as plsc

_NUM_CLASSES = 1000000
_HIDDEN = 64
_BATCH = 16384

_INFO = plsc.get_sparse_core_info()
_NC = _INFO.num_cores
_NS = _INFO.num_subcores
_L = _INFO.num_lanes
_NW = _NC * _NS
_B_PER_W = _BATCH // _NW
_NBLK = (_NUM_CLASSES + 1 + 127) // 128

_mesh = plsc.VectorSubcoreMesh(core_axis_name="c", subcore_axis_name="s")


@functools.partial(
    pl.kernel,
    mesh=_mesh,
    out_type=jax.ShapeDtypeStruct((_BATCH, _HIDDEN), jnp.float32),
    scratch_types=[
        pltpu.SMEM((_B_PER_W,), jnp.int32),
        pltpu.SMEM((_B_PER_W,), jnp.int32),
        pltpu.VMEM_SHARED((_NW, _B_PER_W), jnp.int32),
        pltpu.VMEM_SHARED((_NW, _B_PER_W), jnp.int32),
        pltpu.VMEM((4, _HIDDEN, 128), jnp.float32),
        pltpu.VMEM((_B_PER_W, _HIDDEN), jnp.float32),
        pltpu.SemaphoreType.DMA,
        pltpu.SemaphoreType.DMA,
        pltpu.SemaphoreType.DMA,
        pltpu.SemaphoreType.DMA,
    ],
    compiler_params=pltpu.CompilerParams(needs_layout_passes=False),
)
def _embed(labels_hbm, drop_hbm, tab_t_hbm, tail_hbm, out_hbm,
           lbl_s, drop_s, lbl_sp, drop_sp, ring_v, out_v,
           sem0, sem1, sem2, sem3):
    wid = lax.axis_index("s") * _NC + lax.axis_index("c")
    base = wid * _B_PER_W
    iota = lax.iota(jnp.int32, _L)
    sems = (sem0, sem1, sem2, sem3)

    pltpu.sync_copy(labels_hbm.at[pl.ds(base, _B_PER_W)], lbl_sp.at[wid])
    pltpu.sync_copy(drop_hbm.at[pl.ds(base, _B_PER_W)], drop_sp.at[wid])
    pltpu.sync_copy(lbl_sp.at[wid], lbl_s)
    pltpu.sync_copy(drop_sp.at[wid], drop_s)

    def selected(i):
        return lax.select(drop_s[i] != 0, _NUM_CLASSES, lbl_s[i])

    def issue_fetch(i, par):
        r = selected(i)
        c = r >> 7
        slot = ring_v.at[par]

        @pl.when(c < _NBLK - 1)
        def _main():
            off = pl.multiple_of(c * 128, 128)
            pltpu.async_copy(tab_t_hbm.at[:, pl.ds(off, 128)], slot,
                             sems[par])

        @pl.when(c == _NBLK - 1)
        def _tail():
            pltpu.async_copy(tail_hbm, slot, sems[par])

    def wait_fetch(par):
        pltpu.make_async_copy(tail_hbm, ring_v.at[par], sems[par]).wait()

    def extract(i, par):
        r = selected(i)
        lane = jnp.full((_L,), r & 127, jnp.int32)
        slot = ring_v.at[par]
        orow = out_v.at[i]
        for q in range(_HIDDEN // _L):
            hv = iota + q * _L
            x = plsc.load_gather(slot, [hv, lane])
            orow[pl.ds(q * _L, _L)] = x

    for par in range(4):
        issue_fetch(par, par)

    def body(g, _):
        i = g * 4
        for par in range(4):
            wait_fetch(par)
            extract(i + par, par)

            @pl.when(i + par + 4 < _B_PER_W)
            def _next():
                issue_fetch(i + par + 4, par)

        return 0

    lax.fori_loop(0, _B_PER_W // 4, body, 0)
    pltpu.sync_copy(out_v, out_hbm.at[pl.ds(base, _B_PER_W)])


def kernel(labels, force_drop_ids, embedding_table):
    lbl = labels.astype(jnp.int32)
    drop = force_drop_ids.astype(jnp.int32)
    tab_t = embedding_table.T
    tail_start = (_NBLK - 1) * 128
    tail = jnp.pad(tab_t[:, tail_start:],
                   ((0, 0), (0, _NBLK * 128 - (_NUM_CLASSES + 1))))
    return _embed(lbl, drop, tab_t, tail)


# speedup vs baseline: 42.0753x; 1.1029x over previous
import functools

import jax
import jax.numpy as jnp
from jax import lax
from jax.experimental import pallas as pl
from jax.experimental.pallas import tpu as pltpu
from jax.experimental.pallas import tpu_sc as plsc

_NUM_CLASSES = 1000000
_HIDDEN = 64
_BATCH = 16384

_INFO = plsc.get_sparse_core_info()
_NC = _INFO.num_cores
_NS = _INFO.num_subcores
_L = _INFO.num_lanes
_NW = _NC * _NS
_B_PER_W = _BATCH // _NW
_NBLK = (_NUM_CLASSES + 1 + 127) // 128

_mesh = plsc.VectorSubcoreMesh(core_axis_name="c", subcore_axis_name="s")


@functools.partial(
    pl.kernel,
    mesh=_mesh,
    out_type=jax.ShapeDtypeStruct((_BATCH, _HIDDEN), jnp.float32),
    scratch_types=[
        pltpu.SMEM((_B_PER_W,), jnp.int32),
        pltpu.SMEM((_B_PER_W,), jnp.int32),
        pltpu.VMEM_SHARED((_NW, _B_PER_W), jnp.int32),
        pltpu.VMEM_SHARED((_NW, _B_PER_W), jnp.int32),
        pltpu.VMEM((6, _HIDDEN, 128), jnp.float32),
        pltpu.VMEM((_B_PER_W, _HIDDEN), jnp.float32),
        pltpu.SemaphoreType.DMA,
        pltpu.SemaphoreType.DMA,
        pltpu.SemaphoreType.DMA,
        pltpu.SemaphoreType.DMA,
        pltpu.SemaphoreType.DMA,
        pltpu.SemaphoreType.DMA,
    ],
    compiler_params=pltpu.CompilerParams(needs_layout_passes=False),
)
def _embed(labels_hbm, drop_hbm, tab_t_hbm, tail_hbm, out_hbm,
           lbl_s, drop_s, lbl_sp, drop_sp, ring_v, out_v,
           sem0, sem1, sem2, sem3, sem4, sem5):
    wid = lax.axis_index("s") * _NC + lax.axis_index("c")
    base = wid * _B_PER_W
    iota = lax.iota(jnp.int32, _L)
    sems = (sem0, sem1, sem2, sem3, sem4, sem5)

    pltpu.sync_copy(labels_hbm.at[pl.ds(base, _B_PER_W)], lbl_sp.at[wid])
    pltpu.sync_copy(drop_hbm.at[pl.ds(base, _B_PER_W)], drop_sp.at[wid])
    pltpu.sync_copy(lbl_sp.at[wid], lbl_s)
    pltpu.sync_copy(drop_sp.at[wid], drop_s)

    def selected(i):
        return lax.select(drop_s[i] != 0, _NUM_CLASSES, lbl_s[i])

    def issue_fetch(i, par):
        r = selected(i)
        c = r >> 7
        slot = ring_v.at[par]

        @pl.when(c < _NBLK - 1)
        def _main():
            off = pl.multiple_of(c * 128, 128)
            pltpu.async_copy(tab_t_hbm.at[:, pl.ds(off, 128)], slot,
                             sems[par])

        @pl.when(c == _NBLK - 1)
        def _tail():
            pltpu.async_copy(tail_hbm, slot, sems[par])

    def wait_fetch(par):
        pltpu.make_async_copy(tail_hbm, ring_v.at[par], sems[par]).wait()

    def extract(i, par):
        r = selected(i)
        lane = jnp.full((_L,), r & 127, jnp.int32)
        slot = ring_v.at[par]
        orow = out_v.at[i]
        for q in range(_HIDDEN // _L):
            hv = iota + q * _L
            x = plsc.load_gather(slot, [hv, lane])
            orow[pl.ds(q * _L, _L)] = x

    for par in range(6):
        issue_fetch(par, par)

    def body(g, _):
        i = g * 6
        for par in range(6):
            wait_fetch(par)
            extract(i + par, par)

            @pl.when(i + par + 6 < _B_PER_W)
            def _next():
                issue_fetch(i + par + 6, par)

        return 0

    lax.fori_loop(0, _B_PER_W // 6, body, 0)
    for k in range(_B_PER_W - 6 * (_B_PER_W // 6)):
        i = 6 * (_B_PER_W // 6) + k
        par = i % 6
        wait_fetch(par)
        extract(i, par)
    pltpu.sync_copy(out_v, out_hbm.at[pl.ds(base, _B_PER_W)])


def kernel(labels, force_drop_ids, embedding_table):
    lbl = labels.astype(jnp.int32)
    drop = force_drop_ids.astype(jnp.int32)
    tab_t = embedding_table.T
    tail_start = (_NBLK - 1) * 128
    tail = jnp.pad(tab_t[:, tail_start:],
                   ((0, 0), (0, _NBLK * 128 - (_NUM_CLASSES + 1))))
    return _embed(lbl, drop, tab_t, tail)
